# Initial kernel scaffold; baseline (speedup 1.0000x reference)
import jax
import jax.numpy as jnp
from jax.experimental import pallas as pl


def kernel(z, pos, atomic_mass, embedding, mlp_w1, mlp_b1, mlp_w2, mlp_b2, lin1_w, lin2_w, lin2_b, lin_w, lin_b):
    raise NotImplementedError("write your pallas kernel here")


# speedup vs baseline: 1.4612x; 1.4612x over previous
import functools

import jax
import jax.numpy as jnp
from jax import lax
from jax.experimental import pallas as pl
from jax.experimental.pallas import tpu as pltpu
from jax.experimental.pallas import tpu_sc as plsc

N_NODES = 4096
HIDDEN = 128
NUM_FILTERS = 128
NUM_GAUSSIANS = 50
NODE_CLASS = 120
CUTOFF = 10.0
_DELTA = CUTOFF / (NUM_GAUSSIANS - 1)
_COEFF = -0.5 / _DELTA ** 2
_LOG2 = 0.6931471805599453

_NW = 32
_JPT = N_NODES // _NW
_SLOTS = 96
_E_SLOTS = N_NODES * _SLOTS
_SENTINEL = 1e12
_NPAD = N_NODES + 16
_BLK_B = 2048


def _ssp(x):
    return jnp.maximum(x, 0.0) + jnp.log1p(jnp.exp(-jnp.abs(x))) - _LOG2


def _prep_body(z_ref, pos_ref, am_ref, emb_ref, lin1_ref, xl_ref, post_ref,
               postb_ref, sq_ref):
    z = z_ref[...]
    onehot = (jax.lax.broadcasted_iota(jnp.int32, (N_NODES, NODE_CLASS), 1)
              == z).astype(jnp.float32)
    mass = jax.lax.dot_general(
        onehot, am_ref[...], (((1,), (0,)), ((), ())),
        precision=jax.lax.Precision.HIGHEST,
        preferred_element_type=jnp.float32)
    row = jax.lax.broadcasted_iota(jnp.int32, (N_NODES, 1), 0)
    mass = jnp.where(row < N_NODES - 1, mass, 0.0)
    pos = pos_ref[...]
    num = jnp.sum(mass * pos, axis=0, keepdims=True)
    den = jnp.sum(mass)
    c = num / den
    pos = jnp.where(row == N_NODES - 1, c, pos)
    eye3 = (jax.lax.broadcasted_iota(jnp.int32, (3, 3), 0)
            == jax.lax.broadcasted_iota(jnp.int32, (3, 3), 1)).astype(jnp.float32)
    post = jax.lax.dot_general(eye3, pos, (((1,), (1,)), ((), ())),
                               precision=jax.lax.Precision.HIGHEST,
                               preferred_element_type=jnp.float32)
    post_ref[...] = post
    postb_ref[...] = post.astype(jnp.bfloat16).astype(jnp.float32)
    x2 = post[0:1] * post[0:1]
    y2 = post[1:2] * post[1:2]
    z2 = post[2:3] * post[2:3]
    sq_ref[...] = (x2 + y2) + z2
    e2 = jax.lax.dot_general(emb_ref[...], lin1_ref[...],
                             (((1,), (0,)), ((), ())),
                             precision=jax.lax.Precision.HIGHEST,
                             preferred_element_type=jnp.float32)
    xl_ref[...] = jax.lax.dot_general(onehot, e2, (((1,), (0,)), ((), ())),
                                      precision=jax.lax.Precision.HIGHEST,
                                      preferred_element_type=jnp.float32)


def _edges_body(pose_hbm, src_hbm, d2_hbm, pose_v, src_v, d2_v):
    cid = lax.axis_index("c")
    sid = lax.axis_index("s")
    wid = sid * 2 + cid
    pltpu.sync_copy(pose_hbm, pose_v)
    zero16 = jnp.zeros((16,), jnp.int32)
    sent16 = jnp.full((16,), _SENTINEL, jnp.float32)

    @pl.loop(0, (_JPT * _SLOTS) // 16)
    def _init(k):
        src_v[pl.ds(k * 16, 16)] = zero16
        d2_v[pl.ds(k * 16, 16)] = sent16

    iota16 = lax.iota(jnp.int32, 16)

    @pl.loop(0, _JPT // 16)
    def _jg_loop(jg):
        gbase = wid * _JPT + jg * 16
        gsl = pl.ds(gbase, 16)
        xjv = pose_v[0, gsl]
        yjv = pose_v[1, gsl]
        zjv = pose_v[2, gsl]
        xbjv = pose_v[3, gsl]
        ybjv = pose_v[4, gsl]
        zbjv = pose_v[5, gsl]
        sqjv = pose_v[6, gsl]
        for jl16 in range(16):
            j = gbase + jl16
            xj = xjv[jl16]
            yj = yjv[jl16]
            zj = zjv[jl16]
            xbj = xbjv[jl16]
            ybj = ybjv[jl16]
            zbj = zbjv[jl16]
            sqj = sqjv[jl16]
            base = (jg * 16 + jl16) * _SLOTS
            limit = base + _SLOTS

            def iv_body(iv, off, xj=xj, yj=yj, zj=zj, xbj=xbj, ybj=ybj,
                        zbj=zbj, sqj=sqj, j=j, limit=limit):
                sl = pl.ds(iv * 16, 16)
                xi = pose_v[0, sl]
                yi = pose_v[1, sl]
                zi = pose_v[2, sl]
                xbi = pose_v[3, sl]
                ybi = pose_v[4, sl]
                zbi = pose_v[5, sl]
                sqi = pose_v[6, sl]
                dx = xi - xj
                dy = yi - yj
                dz = zi - zj
                d2e = dx * dx + dy * dy + dz * dz
                pp = xbi * xbj + ybi * ybj + zbi * zbj
                d2a = (sqi + sqj) - 2.0 * pp
                ivec = iv * 16 + iota16
                mask = (d2a < CUTOFF * CUTOFF) & (ivec != j)
                cum = plsc.cumsum(jnp.where(mask, 1, 0))
                positions = off + cum - 1
                mask2 = mask & (positions < limit)
                plsc.store_scatter(src_v, [positions], ivec, mask=mask2)
                plsc.store_scatter(d2_v, [positions], d2e, mask=mask2)
                return off + plsc.all_reduce_population_count(mask2)

            lax.fori_loop(0, N_NODES // 16, iv_body,
                          jnp.full((16,), base, jnp.int32))

    nmy = _JPT * _SLOTS
    pltpu.sync_copy(src_v, src_hbm.at[pl.ds(wid * nmy, nmy)])
    pltpu.sync_copy(d2_v, d2_hbm.at[pl.ds(wid * nmy, nmy)])


def _filter_body(d2_ref, w1_ref, b1_ref, w2_ref, b2_ref, out_ref):
    d2 = d2_ref[...]
    d = jnp.sqrt(d2 + 1e-12)
    offs = (jax.lax.broadcasted_iota(jnp.int32, (1, NUM_GAUSSIANS), 1)
            .astype(jnp.float32) * _DELTA)
    ea = jnp.exp(_COEFF * (d - offs) ** 2)
    h1 = jax.lax.dot_general(ea.astype(jnp.bfloat16), w1_ref[...],
                             (((1,), (0,)), ((), ())),
                             preferred_element_type=jnp.float32) + b1_ref[...]
    a = _ssp(h1)
    w = jax.lax.dot_general(a.astype(jnp.bfloat16), w2_ref[...],
                            (((1,), (0,)), ((), ())),
                            preferred_element_type=jnp.float32) + b2_ref[...]
    cmask = jnp.where(d2 < 1e11,
                      0.5 * (jnp.cos(d * (jnp.pi / CUTOFF)) + 1.0), 0.0)
    out_ref[...] = w * cmask


def _agg_body(xl_hbm, w_hbm, src_hbm, agg_hbm, src_v, xe_v, w_v, agg_v, sem):
    cid = lax.axis_index("c")
    sid = lax.axis_index("s")
    wid = sid * 2 + cid
    nmy = _JPT * _SLOTS
    gbase = wid * nmy
    pltpu.sync_copy(src_hbm.at[pl.ds(gbase, nmy)], src_v)

    @pl.loop(0, _JPT)
    def _j_loop(jl):
        base = jl * _SLOTS
        pltpu.async_copy(xl_hbm.at[src_v.at[pl.ds(base, _SLOTS)]], xe_v,
                         sem).wait()
        pltpu.sync_copy(w_hbm.at[pl.ds(gbase + base, _SLOTS)], w_v)
        zeros = jnp.zeros((16,), jnp.float32)

        def e_body(e, accs):
            return tuple(
                accs[v] + xe_v[e, pl.ds(v * 16, 16)] * w_v[e, pl.ds(v * 16, 16)]
                for v in range(8))

        accs = lax.fori_loop(0, _SLOTS, e_body, (zeros,) * 8)
        for v in range(8):
            agg_v[jl, pl.ds(v * 16, 16)] = accs[v]

    pltpu.sync_copy(agg_v, agg_hbm.at[pl.ds(wid * _JPT, _JPT)])


def _out_body(agg_ref, lin2w_ref, lin2b_ref, linw_ref, linb_ref, o_ref):
    x = jax.lax.dot_general(agg_ref[...], lin2w_ref[...],
                            (((1,), (0,)), ((), ())),
                            precision=jax.lax.Precision.HIGHEST,
                            preferred_element_type=jnp.float32) + lin2b_ref[...]
    x = _ssp(x)
    o_ref[...] = jax.lax.dot_general(x, linw_ref[...], (((1,), (0,)), ((), ())),
                                     precision=jax.lax.Precision.HIGHEST,
                                     preferred_element_type=jnp.float32) + linb_ref[...]


def kernel(z, pos, atomic_mass, embedding, mlp_w1, mlp_b1, mlp_w2, mlp_b2,
           lin1_w, lin2_w, lin2_b, lin_w, lin_b):
    n = N_NODES
    xl, post, postb, sq = pl.pallas_call(
        _prep_body,
        out_shape=(jax.ShapeDtypeStruct((n, NUM_FILTERS), jnp.float32),
                   jax.ShapeDtypeStruct((3, n), jnp.float32),
                   jax.ShapeDtypeStruct((3, n), jnp.float32),
                   jax.ShapeDtypeStruct((1, n), jnp.float32)),
    )(z.reshape(n, 1), pos, atomic_mass.reshape(NODE_CLASS, 1), embedding,
      lin1_w)

    pose = jnp.concatenate([post, postb, sq, jnp.zeros((1, n), jnp.float32)],
                           axis=0)
    pose = jnp.pad(pose, ((0, 0), (0, _NPAD - n)))

    mesh = plsc.VectorSubcoreMesh(core_axis_name="c", subcore_axis_name="s", num_cores=2, num_subcores=16)
    src_e, d2_e = pl.kernel(
        _edges_body,
        out_type=(jax.ShapeDtypeStruct((_E_SLOTS,), jnp.int32),
                  jax.ShapeDtypeStruct((_E_SLOTS,), jnp.float32)),
        mesh=mesh,
        compiler_params=pltpu.CompilerParams(needs_layout_passes=False),
        scratch_types=(pltpu.VMEM((8, _NPAD), jnp.float32),
                       pltpu.VMEM((_JPT * _SLOTS,), jnp.int32),
                       pltpu.VMEM((_JPT * _SLOTS,), jnp.float32)),
    )(pose)

    w_e = pl.pallas_call(
        _filter_body,
        grid=(_E_SLOTS // _BLK_B,),
        in_specs=[
            pl.BlockSpec((_BLK_B, 1), lambda i: (i, 0)),
            pl.BlockSpec((NUM_GAUSSIANS, NUM_FILTERS), lambda i: (0, 0)),
            pl.BlockSpec((1, NUM_FILTERS), lambda i: (0, 0)),
            pl.BlockSpec((NUM_FILTERS, NUM_FILTERS), lambda i: (0, 0)),
            pl.BlockSpec((1, NUM_FILTERS), lambda i: (0, 0)),
        ],
        out_specs=pl.BlockSpec((_BLK_B, NUM_FILTERS), lambda i: (i, 0)),
        out_shape=jax.ShapeDtypeStruct((_E_SLOTS, NUM_FILTERS), jnp.float32),
    )(d2_e.reshape(_E_SLOTS, 1), mlp_w1.astype(jnp.bfloat16),
      mlp_b1.reshape(1, NUM_FILTERS), mlp_w2.astype(jnp.bfloat16),
      mlp_b2.reshape(1, NUM_FILTERS))

    agg = pl.kernel(
        _agg_body,
        out_type=jax.ShapeDtypeStruct((n, NUM_FILTERS), jnp.float32),
        mesh=plsc.VectorSubcoreMesh(core_axis_name="c", subcore_axis_name="s", num_cores=2, num_subcores=16),
        compiler_params=pltpu.CompilerParams(needs_layout_passes=False),
        scratch_types=(pltpu.VMEM((_JPT * _SLOTS,), jnp.int32),
                       pltpu.VMEM((_SLOTS, NUM_FILTERS), jnp.float32),
                       pltpu.VMEM((_SLOTS, NUM_FILTERS), jnp.float32),
                       pltpu.VMEM((_JPT, NUM_FILTERS), jnp.float32),
                       pltpu.SemaphoreType.DMA),
    )(xl, w_e, src_e)

    x = pl.pallas_call(
        _out_body,
        grid=(8,),
        in_specs=[
            pl.BlockSpec((n // 8, NUM_FILTERS), lambda i: (i, 0)),
            pl.BlockSpec((NUM_FILTERS, HIDDEN), lambda i: (0, 0)),
            pl.BlockSpec((1, HIDDEN), lambda i: (0, 0)),
            pl.BlockSpec((HIDDEN, HIDDEN), lambda i: (0, 0)),
            pl.BlockSpec((1, HIDDEN), lambda i: (0, 0)),
        ],
        out_specs=pl.BlockSpec((n // 8, HIDDEN), lambda i: (i, 0)),
        out_shape=jax.ShapeDtypeStruct((n, HIDDEN), jnp.float32),
    )(agg, lin2_w, lin2_b.reshape(1, HIDDEN), lin_w, lin_b.reshape(1, HIDDEN))
    return x
